# Initial kernel scaffold; baseline (speedup 1.0000x reference)
import jax
import jax.numpy as jnp
from jax.experimental import pallas as pl


def kernel(fea, adj, W1, b1, W2, b2):
    raise NotImplementedError("write your pallas kernel here")


# speedup vs baseline: 7.3024x; 7.3024x over previous
import functools

import jax
import jax.numpy as jnp
from jax import lax
from jax.experimental import pallas as pl
from jax.experimental.pallas import tpu as pltpu
from jax.experimental.pallas import tpu_sc as plsc

_NUM_CORES = 2
_NUM_SUBCORES = 16
_NW = _NUM_CORES * _NUM_SUBCORES


def _mm_bias(x, W, b):
  N, K = x.shape
  Do = W.shape[1]
  BR = 512
  G = N // BR

  def body(x_ref, w_ref, b_ref, o_ref):
    o_ref[...] = jnp.dot(x_ref[...], w_ref[...],
                         preferred_element_type=jnp.float32) + b_ref[...]

  return pl.pallas_call(
      body,
      grid=(G,),
      in_specs=[
          pl.BlockSpec((BR, K), lambda i: (i, 0)),
          pl.BlockSpec((K, Do), lambda i: (0, 0)),
          pl.BlockSpec((1, Do), lambda i: (0, 0)),
      ],
      out_specs=pl.BlockSpec((BR, Do), lambda i: (i, 0)),
      out_shape=jax.ShapeDtypeStruct((N, Do), jnp.float32),
  )(x, W, b.reshape(1, Do))


def _mm_bias_sum2(p0, p1, W, b):
  N, K = p0.shape
  Do = W.shape[1]
  BR = 512
  G = N // BR

  def body(p0_ref, p1_ref, w_ref, b_ref, o_ref):
    x = p0_ref[...] + p1_ref[...]
    o_ref[...] = jnp.dot(x, w_ref[...],
                         preferred_element_type=jnp.float32) + b_ref[...]

  return pl.pallas_call(
      body,
      grid=(G,),
      in_specs=[
          pl.BlockSpec((BR, K), lambda i: (i, 0)),
          pl.BlockSpec((BR, K), lambda i: (i, 0)),
          pl.BlockSpec((K, Do), lambda i: (0, 0)),
          pl.BlockSpec((1, Do), lambda i: (0, 0)),
      ],
      out_specs=pl.BlockSpec((BR, Do), lambda i: (i, 0)),
      out_shape=jax.ShapeDtypeStruct((N, Do), jnp.float32),
  )(p0, p1, W, b.reshape(1, Do))


def _add2(p0, p1):
  def body(a_ref, b_ref, o_ref):
    o_ref[...] = a_ref[...] + b_ref[...]

  return pl.pallas_call(
      body,
      out_shape=jax.ShapeDtypeStruct(p0.shape, jnp.float32),
  )(p0, p1)


def _aggregate(sup, src, dst):
  N, D = sup.shape
  E = src.shape[0]
  EP = E // _NW
  C = 80
  NCH = EP // C
  RP = N // _NUM_SUBCORES

  src3 = src.reshape(_NW, NCH, C)
  dst3 = dst.reshape(_NW, NCH, C)
  zeros = jnp.zeros((N, D), jnp.float32)

  mesh = plsc.VectorSubcoreMesh(core_axis_name="c", subcore_axis_name="s")

  @functools.partial(
      pl.kernel,
      mesh=mesh,
      compiler_params=pltpu.CompilerParams(use_tc_tiling_on_sc=False),
      out_type=jax.ShapeDtypeStruct((_NUM_CORES, N, D), jnp.float32),
      scratch_types=[
          pltpu.VMEM((NCH, C), jnp.int32),
          pltpu.VMEM((NCH, C), jnp.int32),
          pltpu.VMEM((C, D), jnp.float32),
          pltpu.VMEM_SHARED((N, D), jnp.float32),
          pltpu.SemaphoreType.DMA,
      ],
  )
  def agg(sup_h, src_h, dst_h, zero_h, out_h, src_v, dst_v, rows_v, acc, sem):
    cid = lax.axis_index("c")
    sid = lax.axis_index("s")
    wid = cid * _NUM_SUBCORES + sid
    pltpu.sync_copy(src_h.at[wid], src_v)
    pltpu.sync_copy(dst_h.at[wid], dst_v)
    r0 = sid * RP
    pltpu.sync_copy(zero_h.at[pl.ds(r0, RP)], acc.at[pl.ds(r0, RP)])
    plsc.subcore_barrier()

    def body(j, carry):
      pltpu.async_copy(sup_h.at[src_v.at[j]], rows_v, sem).wait()
      pltpu.sync_copy(rows_v, acc.at[dst_v.at[j]], add=True)
      return carry

    lax.fori_loop(0, NCH, body, 0)
    plsc.subcore_barrier()
    pltpu.sync_copy(acc.at[pl.ds(r0, RP)], out_h.at[cid, pl.ds(r0, RP)])

  return agg(sup, src3, dst3, zeros)


def kernel(fea, adj, W1, b1, W2, b2):
  N = fea.shape[0]
  Np = ((N + 10239) // 10240) * 10240
  src = adj[0].astype(jnp.int32)
  dst = adj[1].astype(jnp.int32)
  fea_p = jnp.pad(fea, ((0, Np - N), (0, 0)))
  sup1 = _mm_bias(fea_p, W1, b1)
  p1 = _aggregate(sup1, src, dst)
  sup2 = _mm_bias_sum2(p1[0], p1[1], W2, b2)
  p2 = _aggregate(sup2, src, dst)
  return _add2(p2[0], p2[1])[:N]


# speedup vs baseline: 10.9064x; 1.4935x over previous
import functools

import jax
import jax.numpy as jnp
from jax import lax
from jax.experimental import pallas as pl
from jax.experimental.pallas import tpu as pltpu
from jax.experimental.pallas import tpu_sc as plsc

_NUM_CORES = 2
_NUM_SUBCORES = 16
_NW = _NUM_CORES * _NUM_SUBCORES


def _mm_bias(x, W, b):
  N, K = x.shape
  Do = W.shape[1]
  BR = 512
  G = N // BR

  def body(x_ref, w_ref, b_ref, o_ref):
    o_ref[...] = jnp.dot(x_ref[...], w_ref[...],
                         preferred_element_type=jnp.float32) + b_ref[...]

  return pl.pallas_call(
      body,
      grid=(G,),
      in_specs=[
          pl.BlockSpec((BR, K), lambda i: (i, 0)),
          pl.BlockSpec((K, Do), lambda i: (0, 0)),
          pl.BlockSpec((1, Do), lambda i: (0, 0)),
      ],
      out_specs=pl.BlockSpec((BR, Do), lambda i: (i, 0)),
      out_shape=jax.ShapeDtypeStruct((N, Do), jnp.float32),
  )(x, W, b.reshape(1, Do))


def _mm_bias_sum2(p0, p1, W, b):
  N, K = p0.shape
  Do = W.shape[1]
  BR = 512
  G = N // BR

  def body(p0_ref, p1_ref, w_ref, b_ref, o_ref):
    x = p0_ref[...] + p1_ref[...]
    o_ref[...] = jnp.dot(x, w_ref[...],
                         preferred_element_type=jnp.float32) + b_ref[...]

  return pl.pallas_call(
      body,
      grid=(G,),
      in_specs=[
          pl.BlockSpec((BR, K), lambda i: (i, 0)),
          pl.BlockSpec((BR, K), lambda i: (i, 0)),
          pl.BlockSpec((K, Do), lambda i: (0, 0)),
          pl.BlockSpec((1, Do), lambda i: (0, 0)),
      ],
      out_specs=pl.BlockSpec((BR, Do), lambda i: (i, 0)),
      out_shape=jax.ShapeDtypeStruct((N, Do), jnp.float32),
  )(p0, p1, W, b.reshape(1, Do))


def _add2(p0, p1):
  def body(a_ref, b_ref, o_ref):
    o_ref[...] = a_ref[...] + b_ref[...]

  return pl.pallas_call(
      body,
      out_shape=jax.ShapeDtypeStruct(p0.shape, jnp.float32),
  )(p0, p1)


def _aggregate(sup, src, dst):
  N, D = sup.shape
  E = src.shape[0]
  EP = E // _NW
  C = 80
  NCH = EP // C
  RP = N // _NUM_SUBCORES

  src3 = src.reshape(_NW, NCH, C)
  dst3 = dst.reshape(_NW, NCH, C)
  zeros = jnp.zeros((N, D), jnp.float32)

  mesh = plsc.VectorSubcoreMesh(core_axis_name="c", subcore_axis_name="s")

  @functools.partial(
      pl.kernel,
      mesh=mesh,
      compiler_params=pltpu.CompilerParams(use_tc_tiling_on_sc=False),
      out_type=jax.ShapeDtypeStruct((_NUM_CORES, N, D), jnp.float32),
      scratch_types=[
          pltpu.VMEM((NCH, C), jnp.int32),
          pltpu.VMEM((NCH, C), jnp.int32),
          pltpu.VMEM((2, C, D), jnp.float32),
          pltpu.VMEM_SHARED((N, D), jnp.float32),
          pltpu.SemaphoreType.DMA((2,)),
          pltpu.SemaphoreType.DMA((2,)),
      ],
  )
  def agg(sup_h, src_h, dst_h, zero_h, out_h, src_v, dst_v, rows_v, acc,
          gsem, ssem):
    cid = lax.axis_index("c")
    sid = lax.axis_index("s")
    wid = cid * _NUM_SUBCORES + sid
    pltpu.sync_copy(src_h.at[wid], src_v)
    pltpu.sync_copy(dst_h.at[wid], dst_v)
    r0 = sid * RP
    pltpu.sync_copy(zero_h.at[pl.ds(r0, RP)], acc.at[pl.ds(r0, RP)])
    plsc.subcore_barrier()

    def fire_gather(j, p):
      pltpu.async_copy(sup_h.at[src_v.at[j]], rows_v.at[p], gsem.at[p])

    def wait_gather(j, p):
      pltpu.make_async_copy(sup_h.at[src_v.at[j]], rows_v.at[p],
                            gsem.at[p]).wait()

    def fire_scatter(j, p):
      pltpu.async_copy(rows_v.at[p], acc.at[dst_v.at[j]], ssem.at[p],
                       add=True)

    def wait_scatter(j, p):
      pltpu.make_async_copy(rows_v.at[p], acc.at[dst_v.at[j]],
                            ssem.at[p]).wait()

    def body(j, carry):
      p = j % 2
      q = 1 - p

      @pl.when(j >= 2)
      def _():
        wait_scatter(j - 2, p)

      fire_gather(j, p)

      @pl.when(j >= 1)
      def _():
        wait_gather(j - 1, q)
        fire_scatter(j - 1, q)

      return carry

    lax.fori_loop(0, NCH, body, 0)
    pl_ = (NCH - 1) % 2
    wait_gather(NCH - 1, pl_)
    fire_scatter(NCH - 1, pl_)
    wait_scatter(NCH - 2, 1 - pl_)
    wait_scatter(NCH - 1, pl_)
    plsc.subcore_barrier()
    pltpu.sync_copy(acc.at[pl.ds(r0, RP)], out_h.at[cid, pl.ds(r0, RP)])

  return agg(sup, src3, dst3, zeros)


def kernel(fea, adj, W1, b1, W2, b2):
  N = fea.shape[0]
  Np = ((N + 10239) // 10240) * 10240
  src = adj[0].astype(jnp.int32)
  dst = adj[1].astype(jnp.int32)
  fea_p = jnp.pad(fea, ((0, Np - N), (0, 0)))
  sup1 = _mm_bias(fea_p, W1, b1)
  p1 = _aggregate(sup1, src, dst)
  sup2 = _mm_bias_sum2(p1[0], p1[1], W2, b2)
  p2 = _aggregate(sup2, src, dst)
  return _add2(p2[0], p2[1])[:N]


# speedup vs baseline: 12.3705x; 1.1342x over previous
import functools

import jax
import jax.numpy as jnp
from jax import lax
from jax.experimental import pallas as pl
from jax.experimental.pallas import tpu as pltpu
from jax.experimental.pallas import tpu_sc as plsc

_NUM_CORES = 2
_NUM_SUBCORES = 16

_C = 125
_NB = 6
_LAG_G = 1
_LAG_S = 3


def _mm_bias_split(x, W, b):
  N, K = x.shape
  Do = W.shape[1]
  Dc = Do // 2
  BR = 512
  G = N // BR

  def body(x_ref, w_ref, b_ref, o0_ref, o1_ref):
    r = jnp.dot(x_ref[...], w_ref[...],
                preferred_element_type=jnp.float32) + b_ref[...]
    o0_ref[...] = r[:, :Dc]
    o1_ref[...] = r[:, Dc:]

  return pl.pallas_call(
      body,
      grid=(G,),
      in_specs=[
          pl.BlockSpec((BR, K), lambda i: (i, 0)),
          pl.BlockSpec((K, Do), lambda i: (0, 0)),
          pl.BlockSpec((1, Do), lambda i: (0, 0)),
      ],
      out_specs=[
          pl.BlockSpec((BR, Dc), lambda i: (i, 0)),
          pl.BlockSpec((BR, Dc), lambda i: (i, 0)),
      ],
      out_shape=[
          jax.ShapeDtypeStruct((N, Dc), jnp.float32),
          jax.ShapeDtypeStruct((N, Dc), jnp.float32),
      ],
  )(x, W, b.reshape(1, Do))


def _aggregate(sup0, sup1, arr, D):
  N, Dc = sup0.shape
  NCH = arr.shape[1]
  RP = N // _NUM_SUBCORES

  zeros = jnp.zeros((N, Dc), jnp.float32)
  mesh = plsc.VectorSubcoreMesh(core_axis_name="c", subcore_axis_name="s")

  @functools.partial(
      pl.kernel,
      mesh=mesh,
      compiler_params=pltpu.CompilerParams(use_tc_tiling_on_sc=False),
      out_type=jax.ShapeDtypeStruct((N, D), jnp.float32),
      scratch_types=[
          pltpu.VMEM((_NB, 2, _C), jnp.int32),
          pltpu.VMEM((_NB, _C, Dc), jnp.float32),
          pltpu.VMEM_SHARED((N, Dc), jnp.float32),
          pltpu.SemaphoreType.DMA((_NB,)),
          pltpu.SemaphoreType.DMA((_NB,)),
          pltpu.SemaphoreType.DMA((_NB,)),
      ],
  )
  def agg(sup0_h, sup1_h, arr_h, zero_h, out_h, idx_v, rows_v, acc,
          isem, gsem, ssem):
    cid = lax.axis_index("c")
    sid = lax.axis_index("s")
    r0 = sid * RP
    pltpu.sync_copy(zero_h.at[pl.ds(r0, RP)], acc.at[pl.ds(r0, RP)])
    plsc.subcore_barrier()

    def run(sup_h):
      def fire_idx(j, p):
        pltpu.async_copy(arr_h.at[sid, j], idx_v.at[p], isem.at[p])

      def wait_idx(p):
        pltpu.make_async_copy(arr_h.at[sid, 0], idx_v.at[p],
                              isem.at[p]).wait()

      def fire_gather(p):
        pltpu.async_copy(sup_h.at[idx_v.at[p, 0]], rows_v.at[p], gsem.at[p])

      def wait_gather(p):
        pltpu.make_async_copy(sup_h.at[idx_v.at[p, 0]], rows_v.at[p],
                              gsem.at[p]).wait()

      def fire_scatter(p):
        pltpu.async_copy(rows_v.at[p], acc.at[idx_v.at[p, 1]], ssem.at[p],
                         add=True)

      def wait_scatter(p):
        pltpu.make_async_copy(rows_v.at[p], acc.at[idx_v.at[p, 1]],
                              ssem.at[p]).wait()

      def body(j, carry):
        @pl.when(j >= _NB)
        def _():
          wait_scatter(j % _NB)

        fire_idx(j, j % _NB)

        @pl.when(j >= _LAG_G)
        def _():
          wait_idx((j - _LAG_G) % _NB)
          fire_gather((j - _LAG_G) % _NB)

        @pl.when(j >= _LAG_S)
        def _():
          wait_gather((j - _LAG_S) % _NB)
          fire_scatter((j - _LAG_S) % _NB)

        return carry

      lax.fori_loop(0, NCH, body, 0)
      for t in range(NCH - _LAG_G, NCH):
        wait_idx(t % _NB)
        fire_gather(t % _NB)
      for t in range(NCH - _LAG_S, NCH):
        wait_gather(t % _NB)
        fire_scatter(t % _NB)
      for t in range(NCH - _NB, NCH):
        wait_scatter(t % _NB)

    @pl.when(cid == 0)
    def _():
      run(sup0_h)

    @pl.when(cid == 1)
    def _():
      run(sup1_h)

    plsc.subcore_barrier()
    pltpu.sync_copy(acc.at[pl.ds(r0, RP)],
                    out_h.at[pl.ds(r0, RP), pl.ds(cid * Dc, Dc)])

  return agg(sup0, sup1, arr, zeros)


def kernel(fea, adj, W1, b1, W2, b2):
  N = fea.shape[0]
  E = adj.shape[1]
  Np = ((N + 10239) // 10240) * 10240
  EP = E // _NUM_SUBCORES
  NCH = EP // _C
  src = adj[0].astype(jnp.int32).reshape(_NUM_SUBCORES, NCH, _C)
  dst = adj[1].astype(jnp.int32).reshape(_NUM_SUBCORES, NCH, _C)
  arr = jnp.stack([src, dst], axis=2)
  fea_p = jnp.pad(fea, ((0, Np - N), (0, 0)))
  s10, s11 = _mm_bias_split(fea_p, W1, b1)
  x1 = _aggregate(s10, s11, arr, 128)
  s20, s21 = _mm_bias_split(x1, W2, b2)
  out = _aggregate(s20, s21, arr, 64)
  return out[:N]
